# Initial kernel scaffold; baseline (speedup 1.0000x reference)
import jax
import jax.numpy as jnp
from jax.experimental import pallas as pl


def kernel(x, edge_index, batch, W1, b1, W2, b2, W3, b3, Wout, bout):
    raise NotImplementedError("write your pallas kernel here")


# speedup vs baseline: 7.6919x; 7.6919x over previous
import functools

import jax
import jax.numpy as jnp
from jax import lax
from jax.experimental import pallas as pl
from jax.experimental.pallas import tpu as pltpu
from jax.experimental.pallas import tpu_sc as plsc

N = 10000
NP = 10240
E = 160000
EP = 163840
G = 400
GP = 512
D = 256
H = 512
T = 617
TP = 640

NC = 2
NS = 16
LB = 128

@functools.cache
def _mesh():
    return plsc.VectorSubcoreMesh(
        core_axis_name="c", subcore_axis_name="s",
        num_cores=NC, num_subcores=NS)


_f32 = jnp.float32
_i32 = jnp.int32


def _deg_counts_body(dst_hbm, batch_hbm, ones_hbm, zeros_hbm,
                     deg_out, cnt_out,
                     dst_v, bat_v, ones_v, dacc, cacc):
    c = lax.axis_index("c")
    s = lax.axis_index("s")
    w = s * NC + c
    pltpu.sync_copy(dst_hbm.at[w], dst_v)
    pltpu.sync_copy(batch_hbm.at[w], bat_v)
    pltpu.sync_copy(ones_hbm, ones_v)
    for i in range(5):
        pltpu.sync_copy(zeros_hbm, dacc.at[pl.ds(s * 640 + i * 128, 128)])
    pltpu.sync_copy(zeros_hbm.at[pl.ds(0, 32)], cacc.at[pl.ds(s * 32, 32)])
    plsc.subcore_barrier()

    def deg_step(j, carry):
        pltpu.sync_copy(ones_v, dacc.at[dst_v.at[j]], add=True)
        return carry
    lax.fori_loop(0, 40, deg_step, 0)
    for j in range(3):
        pltpu.sync_copy(ones_v, cacc.at[bat_v.at[j]], add=True)
    plsc.subcore_barrier()

    pltpu.sync_copy(dacc.at[pl.ds(s * 640, 640)],
                    deg_out.at[pl.ds(c * NP + s * 640, 640)])
    pltpu.sync_copy(cacc.at[pl.ds(s * 32, 32)],
                    cnt_out.at[pl.ds(c * GP + s * 32, 32)])


@functools.cache
def _deg_counts():
    return pl.kernel(
        _deg_counts_body,
        out_type=(jax.ShapeDtypeStruct((NC * NP, 128), _f32),
                  jax.ShapeDtypeStruct((NC * GP, 128), _f32)),
        mesh=_mesh(),
        scratch_types=[
            pltpu.VMEM((40, 128), _i32),
            pltpu.VMEM((3, 128), _i32),
            pltpu.VMEM((128, 128), _f32),
            pltpu.VMEM_SHARED((NP, 128), _f32),
            pltpu.VMEM_SHARED((GP, 128), _f32),
        ],
    )


def _aggregate_body(nchunks, hs_hbm, src_hbm, dst_hbm, zeros_hbm, out_hbm,
                    src_v, dst_v, buf0, buf1, acc, sem0, sem1):
    c = lax.axis_index("c")
    s = lax.axis_index("s")
    bufs = (buf0, buf1)
    sems = (sem0, sem1)
    nrounds = nchunks // NC

    for r in range(nrounds):
        chunk = r * NC + c
        off = chunk * NP
        for i in range(5):
            pltpu.sync_copy(zeros_hbm, acc.at[pl.ds(s * 640 + i * 128, 128)])
        plsc.subcore_barrier()
        for p in range(2):
            pltpu.sync_copy(src_hbm.at[s * 2 + p], src_v)
            pltpu.sync_copy(dst_hbm.at[s * 2 + p], dst_v)

            def shift_step(j, carry):
                for k in range(4):
                    sl = pl.ds(k * 16, 16)
                    src_v[j, sl] = src_v[j, sl] + off
                return carry
            lax.fori_loop(0, 80, shift_step, 0)

            pltpu.async_copy(hs_hbm.at[src_v.at[0]], buf0, sem0)
            pltpu.async_copy(hs_hbm.at[src_v.at[1]], buf1, sem1)

            def edge_step(g, carry):
                for b in range(2):
                    j = g * 2 + b
                    pltpu.make_async_copy(hs_hbm.at[pl.ds(0, 64)],
                                          bufs[b], sems[b]).wait()
                    pltpu.sync_copy(bufs[b], acc.at[dst_v.at[j]], add=True)

                    @pl.when(j + 2 < 80)
                    def _():
                        pltpu.async_copy(hs_hbm.at[src_v.at[j + 2]],
                                         bufs[b], sems[b])
                return carry
            lax.fori_loop(0, 40, edge_step, 0)
        plsc.subcore_barrier()
        pltpu.sync_copy(acc.at[pl.ds(s * 640, 640)],
                        out_hbm.at[pl.ds(off + s * 640, 640)])
        if r + 1 < nrounds:
            plsc.subcore_barrier()


@functools.cache
def _make_aggregate(nchunks):
    return pl.kernel(
        functools.partial(_aggregate_body, nchunks),
        out_type=jax.ShapeDtypeStruct((nchunks * NP, 128), _f32),
        mesh=_mesh(),
        scratch_types=[
            pltpu.VMEM((80, 64), _i32),
            pltpu.VMEM((80, 64), _i32),
            pltpu.VMEM((64, 128), _f32),
            pltpu.VMEM((64, 128), _f32),
            pltpu.VMEM_SHARED((NP, 128), _f32),
            pltpu.SemaphoreType.DMA,
            pltpu.SemaphoreType.DMA,
        ],
    )


def _pool_body(h3_hbm, batch_hbm, zeros_hbm, sums_out,
               bat_v, buf, acc):
    c = lax.axis_index("c")
    s = lax.axis_index("s")
    pltpu.sync_copy(batch_hbm.at[s], bat_v)
    pltpu.sync_copy(zeros_hbm.at[pl.ds(0, 32)], acc.at[pl.ds(s * 32, 32)])
    plsc.subcore_barrier()
    for r in range(2):
        chunk = r * NC + c
        base = chunk * NP + s * 640
        for k in range(5):
            pltpu.sync_copy(h3_hbm.at[pl.ds(base + k * 128, 128)], buf)
            pltpu.sync_copy(buf, acc.at[bat_v.at[k]], add=True)
        plsc.subcore_barrier()
        pltpu.sync_copy(acc.at[pl.ds(s * 32, 32)],
                        sums_out.at[pl.ds(chunk * GP + s * 32, 32)])
        if r == 0:
            pltpu.sync_copy(zeros_hbm.at[pl.ds(0, 32)],
                            acc.at[pl.ds(s * 32, 32)])
            plsc.subcore_barrier()


@functools.cache
def _pool():
    return pl.kernel(
        _pool_body,
        out_type=jax.ShapeDtypeStruct((4 * GP, 128), _f32),
        mesh=_mesh(),
        scratch_types=[
            pltpu.VMEM((5, 128), _i32),
            pltpu.VMEM((128, 128), _f32),
            pltpu.VMEM_SHARED((GP, 128), _f32),
        ],
    )


_BS = 512


def _scale_x_body(deg_ref, x_ref, dis_ref, hs_ref):
    d = deg_ref[0][:, :1] + deg_ref[1][:, :1]
    dis = lax.rsqrt(jnp.maximum(d, 1.0))
    dis_ref[...] = dis
    for k in range(2):
        hs_ref[k] = x_ref[:, k * 128:(k + 1) * 128] * dis


def _layer_body(cin, relu, scale_out, dis_ref, agg_ref, w_ref, b_ref, out_ref):
    dis = dis_ref[...]
    acc = jnp.zeros((_BS, H), _f32)
    for k in range(cin):
        wk = w_ref[pl.ds(k * 128, 128), :]
        acc = acc + jnp.dot(agg_ref[k] * dis, wk,
                            preferred_element_type=_f32)
    z = acc + b_ref[...]
    if relu:
        z = jnp.maximum(z, 0.0)
    if scale_out:
        z = z * dis
    for k in range(4):
        out_ref[k] = z[:, k * 128:(k + 1) * 128]


def _head_body(cnt_ref, sums_ref, w_ref, b_ref, out_ref):
    cnt = cnt_ref[0][:, :1] + cnt_ref[1][:, :1]
    inv = 1.0 / jnp.maximum(cnt, 1.0)
    acc = jnp.zeros((GP, TP), _f32)
    for k in range(4):
        wk = w_ref[pl.ds(k * 128, 128), :]
        acc = acc + jnp.dot(sums_ref[k] * inv, wk,
                            preferred_element_type=_f32)
    out_ref[...] = acc + b_ref[...]


def _scale_x(deg2, xp):
    grid = NP // _BS
    return pl.pallas_call(
        _scale_x_body,
        grid=(grid,),
        in_specs=[
            pl.BlockSpec((2, _BS, 128), lambda i: (0, i, 0)),
            pl.BlockSpec((_BS, D), lambda i: (i, 0)),
        ],
        out_specs=[
            pl.BlockSpec((_BS, 1), lambda i: (i, 0)),
            pl.BlockSpec((2, _BS, 128), lambda i: (0, i, 0)),
        ],
        out_shape=[
            jax.ShapeDtypeStruct((NP, 1), _f32),
            jax.ShapeDtypeStruct((2, NP, 128), _f32),
        ],
    )(deg2, xp)


def _layer(dis, agg, w, b, cin, relu, scale_out):
    grid = NP // _BS
    return pl.pallas_call(
        functools.partial(_layer_body, cin, relu, scale_out),
        grid=(grid,),
        in_specs=[
            pl.BlockSpec((_BS, 1), lambda i: (i, 0)),
            pl.BlockSpec((cin, _BS, 128), lambda i: (0, i, 0)),
            pl.BlockSpec((cin * 128, H), lambda i: (0, 0)),
            pl.BlockSpec((1, H), lambda i: (0, 0)),
        ],
        out_specs=pl.BlockSpec((4, _BS, 128), lambda i: (0, i, 0)),
        out_shape=jax.ShapeDtypeStruct((4, NP, 128), _f32),
    )(dis, agg, w, b)


def _head(cnt2, sums, wout, bout):
    return pl.pallas_call(
        _head_body,
        in_specs=[
            pl.BlockSpec((2, GP, 128), lambda: (0, 0, 0)),
            pl.BlockSpec((4, GP, 128), lambda: (0, 0, 0)),
            pl.BlockSpec((H, TP), lambda: (0, 0)),
            pl.BlockSpec((1, TP), lambda: (0, 0)),
        ],
        out_specs=pl.BlockSpec((GP, TP), lambda: (0, 0)),
        out_shape=jax.ShapeDtypeStruct((GP, TP), _f32),
    )(cnt2, sums, wout, bout)


def kernel(x, edge_index, batch, W1, b1, W2, b2, W3, b3, Wout, bout):
    src = edge_index[0]
    dst = edge_index[1]
    npad_e = EP - E
    pad_src = (jnp.arange(npad_e, dtype=_i32) % N)
    pad_dst = N + (jnp.arange(npad_e, dtype=_i32) % (NP - N))
    src_p = jnp.concatenate([src, pad_src])
    dst_p = jnp.concatenate([dst, pad_dst])
    srcE = src_p.reshape(32, 80, 64)
    dstE = dst_p.reshape(32, 80, 64)
    dstA = dst_p.reshape(32, 40, 128)

    pad_bA = G + (jnp.arange(32 * 384 - N, dtype=_i32) % (GP - G))
    batA = jnp.concatenate([batch, pad_bA]).reshape(32, 3, 128)
    pad_bE = G + (jnp.arange(NP - N, dtype=_i32) % (GP - G))
    batE = jnp.concatenate([batch, pad_bE]).reshape(16, 5, 128)

    xp = jnp.pad(x, ((0, NP - N), (0, 0)))
    ones128 = jnp.ones((128, 128), _f32)
    zeros128 = jnp.zeros((128, 128), _f32)

    deg2, cnt2 = _deg_counts()(dstA, batA, ones128, zeros128)
    deg2 = deg2.reshape(2, NP, 128)
    cnt2 = cnt2.reshape(2, GP, 128)

    dis, hs0 = _scale_x(deg2, xp)
    hs0 = hs0.reshape(2 * NP, 128)

    agg1 = _make_aggregate(2)(hs0, srcE, dstE, zeros128)
    hs1 = _layer(dis, agg1.reshape(2, NP, 128), W1, b1.reshape(1, H),
                 cin=2, relu=True, scale_out=True)

    agg2 = _make_aggregate(4)(hs1.reshape(4 * NP, 128), srcE, dstE, zeros128)
    hs2 = _layer(dis, agg2.reshape(4, NP, 128), W2, b2.reshape(1, H),
                 cin=4, relu=True, scale_out=True)

    agg3 = _make_aggregate(4)(hs2.reshape(4 * NP, 128), srcE, dstE, zeros128)
    h3 = _layer(dis, agg3.reshape(4, NP, 128), W3, b3.reshape(1, H),
                cin=4, relu=False, scale_out=False)

    sums = _pool()(h3.reshape(4 * NP, 128), batE, zeros128)
    sums = sums.reshape(4, GP, 128)

    wout_p = jnp.pad(Wout, ((0, 0), (0, TP - T)))
    bout_p = jnp.pad(bout, (0, TP - T)).reshape(1, TP)
    out = _head(cnt2, sums, wout_p, bout_p)
    return out[:G, :T]


# speedup vs baseline: 8.7039x; 1.1316x over previous
import functools

import jax
import jax.numpy as jnp
from jax import lax
from jax.experimental import pallas as pl
from jax.experimental.pallas import tpu as pltpu
from jax.experimental.pallas import tpu_sc as plsc

N = 10000
NP = 10240
E = 160000
EP = 163840
G = 400
GP = 512
D = 256
H = 512
T = 617
TP = 640

NC = 2
NS = 16
LB = 128

@functools.cache
def _mesh():
    return plsc.VectorSubcoreMesh(
        core_axis_name="c", subcore_axis_name="s",
        num_cores=NC, num_subcores=NS)


_f32 = jnp.float32
_i32 = jnp.int32


def _deg_counts_body(dst_hbm, batch_hbm, ones_hbm, zeros_hbm,
                     deg_out, cnt_out,
                     dst_v, bat_v, ones_v, dacc, cacc, sem):
    c = lax.axis_index("c")
    s = lax.axis_index("s")
    w = s * NC + c
    pltpu.sync_copy(dst_hbm.at[w], dst_v)
    pltpu.sync_copy(batch_hbm.at[w], bat_v)
    pltpu.sync_copy(ones_hbm, ones_v)
    for i in range(5):
        pltpu.sync_copy(zeros_hbm, dacc.at[pl.ds(s * 640 + i * 128, 128)])
    pltpu.sync_copy(zeros_hbm.at[pl.ds(0, 32)], cacc.at[pl.ds(s * 32, 32)])
    plsc.subcore_barrier()

    def deg_step(j, carry):
        pltpu.async_copy(ones_v, dacc.at[dst_v.at[j]], sem, add=True)
        return carry
    lax.fori_loop(0, 40, deg_step, 0)
    for j in range(3):
        pltpu.async_copy(ones_v, cacc.at[bat_v.at[j]], sem, add=True)

    def drain_step(j, carry):
        pltpu.make_async_copy(ones_v, dacc.at[dst_v.at[0]], sem).wait()
        return carry
    lax.fori_loop(0, 43, drain_step, 0)
    plsc.subcore_barrier()

    pltpu.sync_copy(dacc.at[pl.ds(s * 640, 640)],
                    deg_out.at[pl.ds(c * NP + s * 640, 640)])
    pltpu.sync_copy(cacc.at[pl.ds(s * 32, 32)],
                    cnt_out.at[pl.ds(c * GP + s * 32, 32)])


@functools.cache
def _deg_counts():
    return pl.kernel(
        _deg_counts_body,
        out_type=(jax.ShapeDtypeStruct((NC * NP, 128), _f32),
                  jax.ShapeDtypeStruct((NC * GP, 128), _f32)),
        mesh=_mesh(),
        scratch_types=[
            pltpu.VMEM((40, 128), _i32),
            pltpu.VMEM((3, 128), _i32),
            pltpu.VMEM((128, 128), _f32),
            pltpu.VMEM_SHARED((NP, 128), _f32),
            pltpu.VMEM_SHARED((GP, 128), _f32),
            pltpu.SemaphoreType.DMA,
        ],
    )


def _aggregate_body(nchunks, hs_hbm, src_hbm, dst_hbm, zeros_hbm, out_hbm,
                    src_v, dst_v, buf0, buf1, buf2, acc,
                    gs0, gs1, gs2, ss0, ss1, ss2):
    c = lax.axis_index("c")
    s = lax.axis_index("s")
    bufs = (buf0, buf1, buf2)
    gsem = (gs0, gs1, gs2)
    ssem = (ss0, ss1, ss2)
    nrounds = nchunks // NC

    def wait_gather(u):
        pltpu.make_async_copy(hs_hbm.at[pl.ds(0, 64)],
                              bufs[u], gsem[u]).wait()

    def wait_scatter(u):
        pltpu.make_async_copy(bufs[u], acc.at[dst_v.at[0]], ssem[u]).wait()

    for r in range(nrounds):
        chunk = r * NC + c
        off = chunk * NP
        for i in range(5):
            pltpu.sync_copy(zeros_hbm, acc.at[pl.ds(s * 640 + i * 128, 128)])
        plsc.subcore_barrier()
        for p in range(2):
            pltpu.sync_copy(src_hbm.at[s * 2 + p], src_v)
            pltpu.sync_copy(dst_hbm.at[s * 2 + p], dst_v)

            def shift_step(j, carry):
                for k in range(4):
                    sl = pl.ds(k * 16, 16)
                    src_v[j, sl] = src_v[j, sl] + off
                return carry
            lax.fori_loop(0, 80, shift_step, 0)

            pltpu.async_copy(hs_hbm.at[src_v.at[0]], buf0, gs0)
            pltpu.async_copy(hs_hbm.at[src_v.at[1]], buf1, gs1)

            def edge_step(g, carry):
                for u in range(3):
                    j = g * 3 + u
                    wait_gather(u)
                    pltpu.async_copy(bufs[u], acc.at[dst_v.at[j]],
                                     ssem[u], add=True)
                    un = (u + 2) % 3

                    @pl.when(j >= 1)
                    def _():
                        wait_scatter(un)

                    @pl.when(j + 2 < 80)
                    def _():
                        pltpu.async_copy(hs_hbm.at[src_v.at[j + 2]],
                                         bufs[un], gsem[un])
                return carry
            lax.fori_loop(0, 26, edge_step, 0)
            for j in (78, 79):
                u = j % 3
                wait_gather(u)
                pltpu.async_copy(bufs[u], acc.at[dst_v.at[j]],
                                 ssem[u], add=True)
                wait_scatter((u + 2) % 3)
            wait_scatter(79 % 3)
        plsc.subcore_barrier()
        pltpu.sync_copy(acc.at[pl.ds(s * 640, 640)],
                        out_hbm.at[pl.ds(off + s * 640, 640)])
        if r + 1 < nrounds:
            plsc.subcore_barrier()


@functools.cache
def _make_aggregate(nchunks):
    return pl.kernel(
        functools.partial(_aggregate_body, nchunks),
        out_type=jax.ShapeDtypeStruct((nchunks * NP, 128), _f32),
        mesh=_mesh(),
        scratch_types=[
            pltpu.VMEM((80, 64), _i32),
            pltpu.VMEM((80, 64), _i32),
            pltpu.VMEM((64, 128), _f32),
            pltpu.VMEM((64, 128), _f32),
            pltpu.VMEM((64, 128), _f32),
            pltpu.VMEM_SHARED((NP, 128), _f32),
            pltpu.SemaphoreType.DMA,
            pltpu.SemaphoreType.DMA,
            pltpu.SemaphoreType.DMA,
            pltpu.SemaphoreType.DMA,
            pltpu.SemaphoreType.DMA,
            pltpu.SemaphoreType.DMA,
        ],
    )


def _pool_body(h3_hbm, batch_hbm, zeros_hbm, sums_out,
               bat_v, buf, acc):
    c = lax.axis_index("c")
    s = lax.axis_index("s")
    pltpu.sync_copy(batch_hbm.at[s], bat_v)
    pltpu.sync_copy(zeros_hbm.at[pl.ds(0, 32)], acc.at[pl.ds(s * 32, 32)])
    plsc.subcore_barrier()
    for r in range(2):
        chunk = r * NC + c
        base = chunk * NP + s * 640
        for k in range(5):
            pltpu.sync_copy(h3_hbm.at[pl.ds(base + k * 128, 128)], buf)
            pltpu.sync_copy(buf, acc.at[bat_v.at[k]], add=True)
        plsc.subcore_barrier()
        pltpu.sync_copy(acc.at[pl.ds(s * 32, 32)],
                        sums_out.at[pl.ds(chunk * GP + s * 32, 32)])
        if r == 0:
            pltpu.sync_copy(zeros_hbm.at[pl.ds(0, 32)],
                            acc.at[pl.ds(s * 32, 32)])
            plsc.subcore_barrier()


@functools.cache
def _pool():
    return pl.kernel(
        _pool_body,
        out_type=jax.ShapeDtypeStruct((4 * GP, 128), _f32),
        mesh=_mesh(),
        scratch_types=[
            pltpu.VMEM((5, 128), _i32),
            pltpu.VMEM((128, 128), _f32),
            pltpu.VMEM_SHARED((GP, 128), _f32),
        ],
    )


_BS = 512


def _scale_x_body(deg_ref, x_ref, dis_ref, hs_ref):
    d = deg_ref[0][:, :1] + deg_ref[1][:, :1]
    dis = lax.rsqrt(jnp.maximum(d, 1.0))
    dis_ref[...] = dis
    for k in range(2):
        hs_ref[k] = x_ref[:, k * 128:(k + 1) * 128] * dis


def _layer_body(cin, relu, scale_out, dis_ref, agg_ref, w_ref, b_ref, out_ref):
    dis = dis_ref[...]
    acc = jnp.zeros((_BS, H), _f32)
    for k in range(cin):
        wk = w_ref[pl.ds(k * 128, 128), :]
        acc = acc + jnp.dot(agg_ref[k] * dis, wk,
                            preferred_element_type=_f32)
    z = acc + b_ref[...]
    if relu:
        z = jnp.maximum(z, 0.0)
    if scale_out:
        z = z * dis
    for k in range(4):
        out_ref[k] = z[:, k * 128:(k + 1) * 128]


def _head_body(cnt_ref, sums_ref, w_ref, b_ref, out_ref):
    cnt = cnt_ref[0][:, :1] + cnt_ref[1][:, :1]
    inv = 1.0 / jnp.maximum(cnt, 1.0)
    acc = jnp.zeros((GP, TP), _f32)
    for k in range(4):
        wk = w_ref[pl.ds(k * 128, 128), :]
        acc = acc + jnp.dot(sums_ref[k] * inv, wk,
                            preferred_element_type=_f32)
    out_ref[...] = acc + b_ref[...]


def _scale_x(deg2, xp):
    grid = NP // _BS
    return pl.pallas_call(
        _scale_x_body,
        grid=(grid,),
        in_specs=[
            pl.BlockSpec((2, _BS, 128), lambda i: (0, i, 0)),
            pl.BlockSpec((_BS, D), lambda i: (i, 0)),
        ],
        out_specs=[
            pl.BlockSpec((_BS, 1), lambda i: (i, 0)),
            pl.BlockSpec((2, _BS, 128), lambda i: (0, i, 0)),
        ],
        out_shape=[
            jax.ShapeDtypeStruct((NP, 1), _f32),
            jax.ShapeDtypeStruct((2, NP, 128), _f32),
        ],
    )(deg2, xp)


def _layer(dis, agg, w, b, cin, relu, scale_out):
    grid = NP // _BS
    return pl.pallas_call(
        functools.partial(_layer_body, cin, relu, scale_out),
        grid=(grid,),
        in_specs=[
            pl.BlockSpec((_BS, 1), lambda i: (i, 0)),
            pl.BlockSpec((cin, _BS, 128), lambda i: (0, i, 0)),
            pl.BlockSpec((cin * 128, H), lambda i: (0, 0)),
            pl.BlockSpec((1, H), lambda i: (0, 0)),
        ],
        out_specs=pl.BlockSpec((4, _BS, 128), lambda i: (0, i, 0)),
        out_shape=jax.ShapeDtypeStruct((4, NP, 128), _f32),
    )(dis, agg, w, b)


def _head(cnt2, sums, wout, bout):
    return pl.pallas_call(
        _head_body,
        in_specs=[
            pl.BlockSpec((2, GP, 128), lambda: (0, 0, 0)),
            pl.BlockSpec((4, GP, 128), lambda: (0, 0, 0)),
            pl.BlockSpec((H, TP), lambda: (0, 0)),
            pl.BlockSpec((1, TP), lambda: (0, 0)),
        ],
        out_specs=pl.BlockSpec((GP, TP), lambda: (0, 0)),
        out_shape=jax.ShapeDtypeStruct((GP, TP), _f32),
    )(cnt2, sums, wout, bout)


def kernel(x, edge_index, batch, W1, b1, W2, b2, W3, b3, Wout, bout):
    src = edge_index[0]
    dst = edge_index[1]
    npad_e = EP - E
    pad_src = (jnp.arange(npad_e, dtype=_i32) % N)
    pad_dst = N + (jnp.arange(npad_e, dtype=_i32) % (NP - N))
    src_p = jnp.concatenate([src, pad_src])
    dst_p = jnp.concatenate([dst, pad_dst])
    srcE = src_p.reshape(32, 80, 64)
    dstE = dst_p.reshape(32, 80, 64)
    dstA = dst_p.reshape(32, 40, 128)

    pad_bA = G + (jnp.arange(32 * 384 - N, dtype=_i32) % (GP - G))
    batA = jnp.concatenate([batch, pad_bA]).reshape(32, 3, 128)
    pad_bE = G + (jnp.arange(NP - N, dtype=_i32) % (GP - G))
    batE = jnp.concatenate([batch, pad_bE]).reshape(16, 5, 128)

    xp = jnp.pad(x, ((0, NP - N), (0, 0)))
    ones128 = jnp.ones((128, 128), _f32)
    zeros128 = jnp.zeros((128, 128), _f32)

    deg2, cnt2 = _deg_counts()(dstA, batA, ones128, zeros128)
    deg2 = deg2.reshape(2, NP, 128)
    cnt2 = cnt2.reshape(2, GP, 128)

    dis, hs0 = _scale_x(deg2, xp)
    hs0 = hs0.reshape(2 * NP, 128)

    agg1 = _make_aggregate(2)(hs0, srcE, dstE, zeros128)
    hs1 = _layer(dis, agg1.reshape(2, NP, 128), W1, b1.reshape(1, H),
                 cin=2, relu=True, scale_out=True)

    agg2 = _make_aggregate(4)(hs1.reshape(4 * NP, 128), srcE, dstE, zeros128)
    hs2 = _layer(dis, agg2.reshape(4, NP, 128), W2, b2.reshape(1, H),
                 cin=4, relu=True, scale_out=True)

    agg3 = _make_aggregate(4)(hs2.reshape(4 * NP, 128), srcE, dstE, zeros128)
    h3 = _layer(dis, agg3.reshape(4, NP, 128), W3, b3.reshape(1, H),
                cin=4, relu=False, scale_out=False)

    sums = _pool()(h3.reshape(4 * NP, 128), batE, zeros128)
    sums = sums.reshape(4, GP, 128)

    wout_p = jnp.pad(Wout, ((0, 0), (0, TP - T)))
    bout_p = jnp.pad(bout, (0, TP - T)).reshape(1, TP)
    out = _head(cnt2, sums, wout_p, bout_p)
    return out[:G, :T]
